# baseline (speedup 1.0000x reference)
import functools

import jax
import jax.numpy as jnp
from jax.experimental import pallas as pl

_DIM = 512
_N_HASHES = 256
_BANDWIDTH = 4.0
_N_BUCKETS = 1024


def _lsh_block(x_ref, rv_ref, out_ref):
    proj = jnp.dot(x_ref[...], rv_ref[...], preferred_element_type=jnp.float32)
    out_ref[...] = jnp.floor(proj * (1.0 / _BANDWIDTH)) % _N_BUCKETS


@functools.partial(jax.jit, static_argnames=("block_m",))
def _lsh(x, random_vectors, block_m=1024):
    n = x.shape[0]
    return pl.pallas_call(
        _lsh_block,
        grid=(n // block_m,),
        in_specs=[
            pl.BlockSpec((block_m, _DIM), lambda i: (i, 0)),
            pl.BlockSpec((_DIM, _N_HASHES), lambda i: (0, 0)),
        ],
        out_specs=pl.BlockSpec((block_m, _N_HASHES), lambda i: (i, 0)),
        out_shape=jax.ShapeDtypeStruct((n, _N_HASHES), jnp.float32),
    )(x, random_vectors)


def kernel(x, random_vectors):
    return _lsh(x, random_vectors)


# speedup vs baseline: 1.1262x; 1.1262x over previous
import functools

import jax
import jax.numpy as jnp
from jax.experimental import pallas as pl

_DIM = 512
_N_HASHES = 256
_BANDWIDTH = 4.0
_N_BUCKETS = 1024


def _lsh_block(x_ref, rv_ref, out_ref):
    proj = jnp.dot(x_ref[...], rv_ref[...], preferred_element_type=jnp.float32)
    buckets = jnp.floor(proj * (1.0 / _BANDWIDTH)).astype(jnp.int32) & (_N_BUCKETS - 1)
    out_ref[...] = buckets.astype(jnp.float32)


@functools.partial(jax.jit, static_argnames=("block_m",))
def _lsh(x, random_vectors, block_m=1024):
    n = x.shape[0]
    return pl.pallas_call(
        _lsh_block,
        grid=(n // block_m,),
        in_specs=[
            pl.BlockSpec((block_m, _DIM), lambda i: (i, 0)),
            pl.BlockSpec((_DIM, _N_HASHES), lambda i: (0, 0)),
        ],
        out_specs=pl.BlockSpec((block_m, _N_HASHES), lambda i: (i, 0)),
        out_shape=jax.ShapeDtypeStruct((n, _N_HASHES), jnp.float32),
    )(x, random_vectors)


def kernel(x, random_vectors):
    return _lsh(x, random_vectors)


# speedup vs baseline: 1.4566x; 1.2933x over previous
import functools

import jax
import jax.numpy as jnp
from jax.experimental import pallas as pl

_DIM = 512
_N_HASHES = 256
_BANDWIDTH = 4.0
_N_BUCKETS = 1024


def _lsh_block(x_ref, rv_ref, out_ref):
    proj = jnp.dot(x_ref[...], rv_ref[...], preferred_element_type=jnp.float32)
    buckets = jnp.floor(proj * (1.0 / _BANDWIDTH)).astype(jnp.int32) & (_N_BUCKETS - 1)
    out_ref[...] = buckets.astype(jnp.float32)


@functools.partial(jax.jit, static_argnames=("block_m",))
def _lsh(x, random_vectors, block_m=2048):
    n = x.shape[0]
    return pl.pallas_call(
        _lsh_block,
        grid=(n // block_m,),
        in_specs=[
            pl.BlockSpec((block_m, _DIM), lambda i: (i, 0)),
            pl.BlockSpec((_DIM, _N_HASHES), lambda i: (0, 0)),
        ],
        out_specs=pl.BlockSpec((block_m, _N_HASHES), lambda i: (i, 0)),
        out_shape=jax.ShapeDtypeStruct((n, _N_HASHES), jnp.float32),
    )(x, random_vectors)


def kernel(x, random_vectors):
    return _lsh(x, random_vectors)


# speedup vs baseline: 1.5469x; 1.0620x over previous
import functools

import jax
import jax.numpy as jnp
from jax.experimental import pallas as pl

_DIM = 512
_N_HASHES = 256
_BANDWIDTH = 4.0
_N_BUCKETS = 1024


def _lsh_block(x_ref, rv_ref, out_ref):
    proj = jnp.dot(x_ref[...], rv_ref[...], preferred_element_type=jnp.float32)
    buckets = jnp.floor(proj * (1.0 / _BANDWIDTH)).astype(jnp.int32) & (_N_BUCKETS - 1)
    out_ref[...] = buckets.astype(jnp.float32)


@functools.partial(jax.jit, static_argnames=("block_m",))
def _lsh(x, random_vectors, block_m=4096):
    n = x.shape[0]
    return pl.pallas_call(
        _lsh_block,
        grid=(n // block_m,),
        in_specs=[
            pl.BlockSpec((block_m, _DIM), lambda i: (i, 0)),
            pl.BlockSpec((_DIM, _N_HASHES), lambda i: (0, 0)),
        ],
        out_specs=pl.BlockSpec((block_m, _N_HASHES), lambda i: (i, 0)),
        out_shape=jax.ShapeDtypeStruct((n, _N_HASHES), jnp.float32),
    )(x, random_vectors)


def kernel(x, random_vectors):
    return _lsh(x, random_vectors)


# speedup vs baseline: 1.5608x; 1.0090x over previous
import functools

import jax
import jax.numpy as jnp
from jax.experimental import pallas as pl

_DIM = 512
_N_HASHES = 256
_BANDWIDTH = 4.0
_N_BUCKETS = 1024


def _lsh_block(x_ref, rv_ref, out_ref):
    proj = jnp.dot(x_ref[...], rv_ref[...], preferred_element_type=jnp.float32)
    buckets = jnp.floor(proj * (1.0 / _BANDWIDTH)).astype(jnp.int32) & (_N_BUCKETS - 1)
    out_ref[...] = buckets.astype(jnp.float32)


@functools.partial(jax.jit, static_argnames=("block_m",))
def _lsh(x, random_vectors, block_m=8192):
    n = x.shape[0]
    return pl.pallas_call(
        _lsh_block,
        grid=(n // block_m,),
        in_specs=[
            pl.BlockSpec((block_m, _DIM), lambda i: (i, 0)),
            pl.BlockSpec((_DIM, _N_HASHES), lambda i: (0, 0)),
        ],
        out_specs=pl.BlockSpec((block_m, _N_HASHES), lambda i: (i, 0)),
        out_shape=jax.ShapeDtypeStruct((n, _N_HASHES), jnp.float32),
    )(x, random_vectors)


def kernel(x, random_vectors):
    return _lsh(x, random_vectors)
